# baseline (speedup 1.0000x reference)
import functools

import jax
import jax.numpy as jnp
from jax import lax
from jax.experimental import pallas as pl
from jax.experimental.pallas import tpu as pltpu
from jax.experimental.pallas import tpu_sc as plsc

N_NODES = 10000
N_EDGES = 320000
NODE_TYPES = 9
LATENT = 128

EB = 2000
NB = 2000

NC = 2
NS = 16
NW = NC * NS
NPH = 5
EPH = N_EDGES // NPH
EPW = EPH // NW
CH = 40
NCHUNK = EPW // CH
NPAIR = NCHUNK // 2
NPAD = 10112
NPS = NPAD // NS
EBH = EPH // 80

_f32 = jnp.float32


def _ln(y, scale, bias):
    mu = jnp.mean(y, axis=-1, keepdims=True)
    var = jnp.mean((y - mu) * (y - mu), axis=-1, keepdims=True)
    return (y - mu) / jnp.sqrt(var + 1e-5) * scale + bias


def _dot(a, b):
    return jnp.dot(a, b, preferred_element_type=_f32)


def _bf(x):
    return x.astype(jnp.bfloat16).astype(_f32)


def _node_enc_kernel(vel_ref, nt_ref, wv_ref, woh_ref, b1_ref, w2_ref, b2_ref,
                     w3_ref, b3_ref, lns_ref, lnb_ref, out_ref):
    vel = _bf(vel_ref[...])
    t = nt_ref[...]
    wv = _bf(wv_ref[...])
    woh = _bf(woh_ref[...])
    h = vel[:, 0:1] * wv[0:1, :] + vel[:, 1:2] * wv[1:2, :]
    for k in range(NODE_TYPES):
        h = h + jnp.where(t == float(k), 1.0, 0.0) * woh[k:k + 1, :]
    h = h + b1_ref[...]
    h = jnp.maximum(h, 0.0)
    h = jnp.maximum(_dot(h, w2_ref[...]) + b2_ref[...], 0.0)
    y = _dot(h, w3_ref[...]) + b3_ref[...]
    out_ref[...] = _ln(y, lns_ref[...], lnb_ref[...])


def _edge_enc_kernel(ef_ref, wf_ref, b1_ref, w2_ref, b2_ref, w3_ref, b3_ref,
                     lns_ref, lnb_ref, out_ref):
    ef = _bf(ef_ref[...])
    wf = _bf(wf_ref[...])
    h = (ef[:, 0:1] * wf[0:1, :] + ef[:, 1:2] * wf[1:2, :]
         + ef[:, 2:3] * wf[2:3, :] + b1_ref[...])
    h = jnp.maximum(h, 0.0)
    h = jnp.maximum(_dot(h, w2_ref[...]) + b2_ref[...], 0.0)
    y = _dot(h, w3_ref[...]) + b3_ref[...]
    out_ref[...] = _ln(y, lns_ref[...], lnb_ref[...])


def _edge_step_kernel(el_ref, g1_ref, g2_ref, w1_ref, b1_ref, w2_ref, b2_ref,
                      w3_ref, b3_ref, lns_ref, lnb_ref, ne_ref, elo_ref):
    x = el_ref[...]
    xin = jnp.concatenate([x, g1_ref[...], g2_ref[...]], axis=-1)
    h = _dot(xin, w1_ref[...]) + b1_ref[...]
    h = jnp.maximum(h, 0.0)
    h = jnp.maximum(_dot(h, w2_ref[...]) + b2_ref[...], 0.0)
    y = _dot(h, w3_ref[...]) + b3_ref[...]
    y = _ln(y, lns_ref[...], lnb_ref[...])
    ne_ref[...] = y
    elo_ref[...] = x + y


def _node_step_kernel(nl_ref, agg0_ref, agg1_ref, agg2_ref, agg3_ref,
                      agg4_ref, w1_ref, b1_ref, w2_ref, b2_ref, w3_ref,
                      b3_ref, lns_ref, lnb_ref, out_ref):
    nl = nl_ref[...]
    agg = ((agg0_ref[0] + agg0_ref[1]) + (agg1_ref[0] + agg1_ref[1])
           + (agg2_ref[0] + agg2_ref[1]) + (agg3_ref[0] + agg3_ref[1])
           + (agg4_ref[0] + agg4_ref[1]))
    nin = jnp.concatenate([nl, agg], axis=-1)
    h = _dot(nin, w1_ref[...]) + b1_ref[...]
    h = jnp.maximum(h, 0.0)
    h = jnp.maximum(_dot(h, w2_ref[...]) + b2_ref[...], 0.0)
    y = _dot(h, w3_ref[...]) + b3_ref[...]
    out_ref[...] = nl + _ln(y, lns_ref[...], lnb_ref[...])


def _decoder_kernel(nl_ref, w1_ref, b1_ref, w2_ref, b2_ref, w3_ref, b3_ref,
                    out_ref):
    h = jnp.maximum(_dot(nl_ref[...], w1_ref[...]) + b1_ref[...], 0.0)
    h = jnp.maximum(_dot(h, w2_ref[...]) + b2_ref[...], 0.0)
    out_ref[...] = _dot(h, w3_ref[...]) + b3_ref[...]


def _sc_mesh():
    return plsc.VectorSubcoreMesh(core_axis_name="c", subcore_axis_name="s")


def _sc_gather_pair(table, sidx3d, didx3d, phase):
    @functools.partial(
        pl.kernel,
        out_type=(jax.ShapeDtypeStruct((N_EDGES, LATENT), _f32),
                  jax.ShapeDtypeStruct((N_EDGES, LATENT), _f32)),
        mesh=_sc_mesh(),
        name=f"sc_gather_p{phase}",
        scratch_types=[
            pltpu.VMEM((NCHUNK, CH), jnp.int32),
            pltpu.VMEM((NCHUNK, CH), jnp.int32),
            pltpu.VMEM((CH, LATENT), _f32),
            pltpu.VMEM((CH, LATENT), _f32),
            pltpu.VMEM((CH, LATENT), _f32),
            pltpu.VMEM((CH, LATENT), _f32),
            pltpu.SemaphoreType.DMA,
            pltpu.SemaphoreType.DMA,
            pltpu.SemaphoreType.DMA,
            pltpu.SemaphoreType.DMA,
        ],
    )
    def k(table_hbm, sidx_hbm, didx_hbm, g1_hbm, g2_hbm,
          sidx_v, didx_v, sbufa, sbufb, dbufa, dbufb,
          ssema, ssemb, dsema, dsemb):
        wid = lax.axis_index("s") * NC + lax.axis_index("c")
        base = phase * EPH + wid * EPW
        pltpu.sync_copy(sidx_hbm.at[wid], sidx_v)
        pltpu.sync_copy(didx_hbm.at[wid], didx_v)

        def gth(idx_v, j, buf, sem):
            pltpu.async_copy(table_hbm.at[idx_v.at[j]], buf, sem)

        def gwait(idx_v, j, buf, sem):
            pltpu.make_async_copy(table_hbm.at[idx_v.at[j]], buf, sem).wait()

        def wb(j, buf, sem, out_hbm):
            pltpu.async_copy(buf, out_hbm.at[pl.ds(base + j * CH, CH)], sem)

        def wbwait(j, buf, sem, out_hbm):
            pltpu.make_async_copy(
                buf, out_hbm.at[pl.ds(base + j * CH, CH)], sem).wait()

        gth(sidx_v, 0, sbufa, ssema)
        gth(didx_v, 0, dbufa, dsema)
        gth(sidx_v, 1, sbufb, ssemb)
        gth(didx_v, 1, dbufb, dsemb)
        gwait(sidx_v, 0, sbufa, ssema)
        wb(0, sbufa, ssema, g1_hbm)
        gwait(didx_v, 0, dbufa, dsema)
        wb(0, dbufa, dsema, g2_hbm)
        gwait(sidx_v, 1, sbufb, ssemb)
        wb(1, sbufb, ssemb, g1_hbm)
        gwait(didx_v, 1, dbufb, dsemb)
        wb(1, dbufb, dsemb, g2_hbm)

        def body(i, carry):
            j0 = 2 * i
            j1 = 2 * i + 1
            wbwait(j0 - 2, sbufa, ssema, g1_hbm)
            gth(sidx_v, j0, sbufa, ssema)
            wbwait(j0 - 2, dbufa, dsema, g2_hbm)
            gth(didx_v, j0, dbufa, dsema)
            wbwait(j1 - 2, sbufb, ssemb, g1_hbm)
            gth(sidx_v, j1, sbufb, ssemb)
            wbwait(j1 - 2, dbufb, dsemb, g2_hbm)
            gth(didx_v, j1, dbufb, dsemb)
            gwait(sidx_v, j0, sbufa, ssema)
            wb(j0, sbufa, ssema, g1_hbm)
            gwait(didx_v, j0, dbufa, dsema)
            wb(j0, dbufa, dsema, g2_hbm)
            gwait(sidx_v, j1, sbufb, ssemb)
            wb(j1, sbufb, ssemb, g1_hbm)
            gwait(didx_v, j1, dbufb, dsemb)
            wb(j1, dbufb, dsemb, g2_hbm)
            return carry

        lax.fori_loop(1, NPAIR, body, 0)

        last = 2 * NPAIR - 2
        wbwait(last, sbufa, ssema, g1_hbm)
        wbwait(last, dbufa, dsema, g2_hbm)
        wbwait(last + 1, sbufb, ssemb, g1_hbm)
        wbwait(last + 1, dbufb, dsemb, g2_hbm)
        if NCHUNK % 2 == 1:
            jlast = NCHUNK - 1
            gth(sidx_v, jlast, sbufa, ssema)
            gth(didx_v, jlast, dbufa, dsema)
            gwait(sidx_v, jlast, sbufa, ssema)
            pltpu.sync_copy(sbufa, g1_hbm.at[pl.ds(base + jlast * CH, CH)])
            gwait(didx_v, jlast, dbufa, dsema)
            pltpu.sync_copy(dbufa, g2_hbm.at[pl.ds(base + jlast * CH, CH)])

    return k(table, sidx3d, didx3d)


def _sc_scatter_add(new_edge, didx3d, zeros_stripe, phase):
    @functools.partial(
        pl.kernel,
        out_type=jax.ShapeDtypeStruct((NC, NPAD, LATENT), _f32),
        mesh=_sc_mesh(),
        name=f"sc_scatter_p{phase}",
        scratch_types=[
            pltpu.VMEM((NCHUNK, CH), jnp.int32),
            pltpu.VMEM((CH, LATENT), _f32),
            pltpu.VMEM((CH, LATENT), _f32),
            pltpu.VMEM_SHARED((NPAD, LATENT), _f32),
            pltpu.SemaphoreType.DMA,
            pltpu.SemaphoreType.DMA,
        ],
    )
    def k(ne_hbm, didx_hbm, zero_hbm, out_hbm, idx_v, rowa, rowb, acc,
          sema, semb):
        cid = lax.axis_index("c")
        sid = lax.axis_index("s")
        wid = sid * NC + cid
        base = phase * EPH + wid * EPW
        pltpu.sync_copy(didx_hbm.at[wid], idx_v)
        pltpu.sync_copy(zero_hbm, acc.at[pl.ds(sid * NPS, NPS)])
        plsc.subcore_barrier()

        def load(j, buf, sem):
            pltpu.async_copy(ne_hbm.at[pl.ds(base + j * CH, CH)], buf, sem)

        def lwait(j, buf, sem):
            pltpu.make_async_copy(
                ne_hbm.at[pl.ds(base + j * CH, CH)], buf, sem).wait()

        def sadd(j, buf, sem):
            pltpu.async_copy(buf, acc.at[idx_v.at[j]], sem, add=True)

        def swait(j, buf, sem):
            pltpu.make_async_copy(buf, acc.at[idx_v.at[j]], sem).wait()

        load(0, rowa, sema)
        load(1, rowb, semb)
        lwait(0, rowa, sema)
        sadd(0, rowa, sema)
        lwait(1, rowb, semb)
        sadd(1, rowb, semb)

        def body(i, carry):
            j0 = 2 * i
            j1 = 2 * i + 1
            swait(j0 - 2, rowa, sema)
            load(j0, rowa, sema)
            swait(j1 - 2, rowb, semb)
            load(j1, rowb, semb)
            lwait(j0, rowa, sema)
            sadd(j0, rowa, sema)
            lwait(j1, rowb, semb)
            sadd(j1, rowb, semb)
            return carry

        lax.fori_loop(1, NPAIR, body, 0)
        last = 2 * NPAIR - 2
        swait(last, rowa, sema)
        swait(last + 1, rowb, semb)
        if NCHUNK % 2 == 1:
            jlast = NCHUNK - 1
            load(jlast, rowa, sema)
            lwait(jlast, rowa, sema)
            pltpu.sync_copy(rowa, acc.at[idx_v.at[jlast]], add=True)

        plsc.subcore_barrier()
        pltpu.sync_copy(acc.at[pl.ds(sid * NPS, NPS)],
                        out_hbm.at[cid, pl.ds(sid * NPS, NPS)])

    return k(new_edge, didx3d, zeros_stripe)


def _full(shape):
    nd = len(shape)
    return pl.BlockSpec(shape, lambda i: (0,) * nd)


def _rows(block, width):
    return pl.BlockSpec((block, width), lambda i: (i, 0))


def _rows_off(block, width, off):
    return pl.BlockSpec((block, width), lambda i, _o=off: (i + _o, 0))


def _call(body, grid, in_specs, out_specs, out_shape, args):
    return pl.pallas_call(
        body,
        grid=(grid,),
        in_specs=in_specs,
        out_specs=out_specs,
        out_shape=out_shape,
    )(*args)


def _w(x):
    return _full(x.shape)


def kernel(velocity, mesh_pos, params, node_type, srcs, dsts):
    p = params

    nm, ns = p["node_norm_mean"], p["node_norm_std"]
    ne_l = p["node_enc"]["layers"]
    w1n = ne_l[0]["W"] / ns[:, None]
    b1n = ne_l[0]["b"] - nm @ w1n
    wv = w1n[:2]
    woh = w1n[2:]

    em, es = p["edge_norm_mean"], p["edge_norm_std"]
    ee_l = p["edge_enc"]["layers"]
    w1e = ee_l[0]["W"] / es[:, None]
    b1e = ee_l[0]["b"] - em @ w1e

    dec_l = p["decoder"]["layers"]
    ostd, omean = p["out_norm_std"], p["out_norm_mean"]
    wd3 = dec_l[2]["W"] * ostd[None, :]
    bd3 = dec_l[2]["b"] * ostd + omean
    wd3p = jnp.zeros((LATENT, 128), _f32).at[:, :2].set(wd3)
    bd3p = jnp.zeros((128,), _f32).at[:2].set(bd3)

    def row(v):
        return v.reshape(1, -1)

    rel = mesh_pos[srcs, :] - mesh_pos[dsts, :]
    rel_norm = jnp.sqrt(jnp.sum(rel * rel, axis=-1, keepdims=True) + 1e-12)
    ef = jnp.concatenate([rel, rel_norm,
                          jnp.zeros((N_EDGES, 1), _f32)], axis=-1)

    ntf = node_type.astype(_f32).reshape(N_NODES, 1)
    node_lat = _call(
        _node_enc_kernel, N_NODES // NB,
        [_rows(NB, 2), _rows(NB, 1), _w(wv), _w(woh), _full((1, LATENT)),
         _w(ne_l[1]["W"]), _full((1, LATENT)), _w(ne_l[2]["W"]),
         _full((1, LATENT)), _full((1, LATENT)), _full((1, LATENT))],
        _rows(NB, LATENT),
        jax.ShapeDtypeStruct((N_NODES, LATENT), _f32),
        (velocity, ntf, wv, woh, row(b1n), ne_l[1]["W"], row(ne_l[1]["b"]),
         ne_l[2]["W"], row(ne_l[2]["b"]), row(p["node_enc"]["ln_scale"]),
         row(p["node_enc"]["ln_bias"])))

    edge_lat = _call(
        _edge_enc_kernel, N_EDGES // EB,
        [_rows(EB, 4), _w(w1e), _full((1, LATENT)), _w(ee_l[1]["W"]),
         _full((1, LATENT)), _w(ee_l[2]["W"]), _full((1, LATENT)),
         _full((1, LATENT)), _full((1, LATENT))],
        _rows(EB, LATENT),
        jax.ShapeDtypeStruct((N_EDGES, LATENT), _f32),
        (ef, w1e, row(b1e), ee_l[1]["W"], row(ee_l[1]["b"]), ee_l[2]["W"],
         row(ee_l[2]["b"]), row(p["edge_enc"]["ln_scale"]),
         row(p["edge_enc"]["ln_bias"])))

    sidx4d = srcs.astype(jnp.int32).reshape(NPH, NW, NCHUNK, CH)
    didx4d = dsts.astype(jnp.int32).reshape(NPH, NW, NCHUNK, CH)
    sidx = [sidx4d[ph] for ph in range(NPH)]
    didx = [didx4d[ph] for ph in range(NPH)]
    zeros_stripe = jnp.zeros((NPS, LATENT), _f32)

    edge_lat_ph = [edge_lat] * NPH
    for step in p["mp"]:
        el_ = step["edge"]["layers"]
        nl_ = step["node"]["layers"]
        w1 = el_[0]["W"]

        gs = [_sc_gather_pair(node_lat, sidx[ph], didx[ph], ph)
              for ph in range(NPH)]

        aggs = []
        new_el_ph = []
        for ph in range(NPH):
            g1, g2 = gs[ph]
            off = ph * (EPH // EB)
            ne_el = _call(
                _edge_step_kernel, EPH // EB,
                [_rows_off(EB, LATENT, off), _rows_off(EB, LATENT, off),
                 _rows_off(EB, LATENT, off),
                 _w(w1), _full((1, LATENT)), _w(el_[1]["W"]),
                 _full((1, LATENT)), _w(el_[2]["W"]), _full((1, LATENT)),
                 _full((1, LATENT)), _full((1, LATENT))],
                [_rows_off(EB, LATENT, off), _rows_off(EB, LATENT, off)],
                [jax.ShapeDtypeStruct((N_EDGES, LATENT), _f32),
                 jax.ShapeDtypeStruct((N_EDGES, LATENT), _f32)],
                (edge_lat_ph[ph], g1, g2, w1, row(el_[0]["b"]), el_[1]["W"],
                 row(el_[1]["b"]), el_[2]["W"], row(el_[2]["b"]),
                 row(step["edge"]["ln_scale"]), row(step["edge"]["ln_bias"])))
            new_edge, elo = ne_el
            new_el_ph.append(elo)
            aggs.append(_sc_scatter_add(new_edge, didx[ph], zeros_stripe, ph))
        edge_lat_ph = new_el_ph

        wn1 = nl_[0]["W"]
        node_lat = _call(
            _node_step_kernel, N_NODES // NB,
            [_rows(NB, LATENT)]
            + [pl.BlockSpec((NC, NB, LATENT), lambda i: (0, i, 0))
               for _ in range(NPH)]
            + [_w(wn1), _full((1, LATENT)), _w(nl_[1]["W"]),
               _full((1, LATENT)), _w(nl_[2]["W"]), _full((1, LATENT)),
               _full((1, LATENT)), _full((1, LATENT))],
            _rows(NB, LATENT),
            jax.ShapeDtypeStruct((N_NODES, LATENT), _f32),
            tuple([node_lat] + aggs + [wn1, row(nl_[0]["b"]), nl_[1]["W"],
             row(nl_[1]["b"]), nl_[2]["W"], row(nl_[2]["b"]),
             row(step["node"]["ln_scale"]), row(step["node"]["ln_bias"])]))

    outp = _call(
        _decoder_kernel, N_NODES // NB,
        [_rows(NB, LATENT), _w(dec_l[0]["W"]), _full((1, LATENT)),
         _w(dec_l[1]["W"]), _full((1, LATENT)), _w(wd3p), _full((1, 128))],
        _rows(NB, 128),
        jax.ShapeDtypeStruct((N_NODES, 128), _f32),
        (node_lat, dec_l[0]["W"], row(dec_l[0]["b"]), dec_l[1]["W"],
         row(dec_l[1]["b"]), wd3p, row(bd3p)))
    return outp[:, :2]


# speedup vs baseline: 1.1690x; 1.1690x over previous
import functools

import jax
import jax.numpy as jnp
from jax import lax
from jax.experimental import pallas as pl
from jax.experimental.pallas import tpu as pltpu
from jax.experimental.pallas import tpu_sc as plsc

N_NODES = 10000
N_EDGES = 320000
NODE_TYPES = 9
LATENT = 128

EB = 2000
NB = 2000

NC = 2
NS = 16
NW = NC * NS
NPH = 2
EPH = N_EDGES // NPH
EPW = EPH // NW
CH = 40
NCHUNK = EPW // CH
NPAIR = NCHUNK // 2
NPAD = 10112
NPS = NPAD // NS
EBH = EPH // 80

_f32 = jnp.float32


def _ln(y, scale, bias):
    mu = jnp.mean(y, axis=-1, keepdims=True)
    var = jnp.mean((y - mu) * (y - mu), axis=-1, keepdims=True)
    return (y - mu) / jnp.sqrt(var + 1e-5) * scale + bias


def _dot(a, b):
    return jnp.dot(a, b, preferred_element_type=_f32)


def _bf(x):
    return x.astype(jnp.bfloat16).astype(_f32)


def _node_enc_kernel(vel_ref, nt_ref, wv_ref, woh_ref, b1_ref, w2_ref, b2_ref,
                     w3_ref, b3_ref, lns_ref, lnb_ref, out_ref):
    vel = _bf(vel_ref[...])
    t = nt_ref[...]
    wv = _bf(wv_ref[...])
    woh = _bf(woh_ref[...])
    h = vel[:, 0:1] * wv[0:1, :] + vel[:, 1:2] * wv[1:2, :]
    for k in range(NODE_TYPES):
        h = h + jnp.where(t == float(k), 1.0, 0.0) * woh[k:k + 1, :]
    h = h + b1_ref[...]
    h = jnp.maximum(h, 0.0)
    h = jnp.maximum(_dot(h, w2_ref[...]) + b2_ref[...], 0.0)
    y = _dot(h, w3_ref[...]) + b3_ref[...]
    out_ref[...] = _ln(y, lns_ref[...], lnb_ref[...])


def _edge_enc_kernel(s16_ref, d16_ref, wf_ref, b1_ref, w2_ref, b2_ref,
                     w3_ref, b3_ref, lns_ref, lnb_ref, out_ref):
    sxy = s16_ref[...]
    dxy = d16_ref[...]
    rx = sxy[:, 0:1] - dxy[:, 0:1]
    ry = sxy[:, 1:2] - dxy[:, 1:2]
    norm = jnp.sqrt(rx * rx + ry * ry + 1e-12)
    wf = _bf(wf_ref[...])
    h = (_bf(rx) * wf[0:1, :] + _bf(ry) * wf[1:2, :]
         + _bf(norm) * wf[2:3, :] + b1_ref[...])
    h = jnp.maximum(h, 0.0)
    h = jnp.maximum(_dot(h, w2_ref[...]) + b2_ref[...], 0.0)
    y = _dot(h, w3_ref[...]) + b3_ref[...]
    out_ref[...] = _ln(y, lns_ref[...], lnb_ref[...])


def _edge_step_kernel(el_ref, g1_ref, g2_ref, w1_ref, b1_ref, w2_ref, b2_ref,
                      w3_ref, b3_ref, lns_ref, lnb_ref, ne_ref, elo_ref):
    x = el_ref[...]
    xin = jnp.concatenate([x, g1_ref[...], g2_ref[...]], axis=-1)
    h = _dot(xin, w1_ref[...]) + b1_ref[...]
    h = jnp.maximum(h, 0.0)
    h = jnp.maximum(_dot(h, w2_ref[...]) + b2_ref[...], 0.0)
    y = _dot(h, w3_ref[...]) + b3_ref[...]
    y = _ln(y, lns_ref[...], lnb_ref[...])
    ne_ref[...] = y
    elo_ref[...] = x + y


def _node_step_kernel(nl_ref, agg0_ref, agg1_ref, w1_ref, b1_ref, w2_ref,
                      b2_ref, w3_ref, b3_ref, lns_ref, lnb_ref, out_ref):
    nl = nl_ref[...]
    agg = (agg0_ref[0] + agg0_ref[1]) + (agg1_ref[0] + agg1_ref[1])
    nin = jnp.concatenate([nl, agg], axis=-1)
    h = _dot(nin, w1_ref[...]) + b1_ref[...]
    h = jnp.maximum(h, 0.0)
    h = jnp.maximum(_dot(h, w2_ref[...]) + b2_ref[...], 0.0)
    y = _dot(h, w3_ref[...]) + b3_ref[...]
    out_ref[...] = nl + _ln(y, lns_ref[...], lnb_ref[...])


def _decoder_kernel(nl_ref, w1_ref, b1_ref, w2_ref, b2_ref, w3_ref, b3_ref,
                    out_ref):
    h = jnp.maximum(_dot(nl_ref[...], w1_ref[...]) + b1_ref[...], 0.0)
    h = jnp.maximum(_dot(h, w2_ref[...]) + b2_ref[...], 0.0)
    out_ref[...] = _dot(h, w3_ref[...]) + b3_ref[...]


def _sc_mesh():
    return plsc.VectorSubcoreMesh(core_axis_name="c", subcore_axis_name="s")


def _sc_gather_pair(table, sidx3d, didx3d, phase, width=LATENT):
    @functools.partial(
        pl.kernel,
        out_type=(jax.ShapeDtypeStruct((N_EDGES, width), _f32),
                  jax.ShapeDtypeStruct((N_EDGES, width), _f32)),
        mesh=_sc_mesh(),
        name=f"sc_gather_p{phase}_w{width}",
        scratch_types=[
            pltpu.VMEM((NCHUNK, CH), jnp.int32),
            pltpu.VMEM((NCHUNK, CH), jnp.int32),
            pltpu.VMEM((CH, width), _f32),
            pltpu.VMEM((CH, width), _f32),
            pltpu.VMEM((CH, width), _f32),
            pltpu.VMEM((CH, width), _f32),
            pltpu.SemaphoreType.DMA,
            pltpu.SemaphoreType.DMA,
            pltpu.SemaphoreType.DMA,
            pltpu.SemaphoreType.DMA,
        ],
    )
    def k(table_hbm, sidx_hbm, didx_hbm, g1_hbm, g2_hbm,
          sidx_v, didx_v, sbufa, sbufb, dbufa, dbufb,
          ssema, ssemb, dsema, dsemb):
        wid = lax.axis_index("s") * NC + lax.axis_index("c")
        base = phase * EPH + wid * EPW
        pltpu.sync_copy(sidx_hbm.at[wid], sidx_v)
        pltpu.sync_copy(didx_hbm.at[wid], didx_v)

        def gth(idx_v, j, buf, sem):
            pltpu.async_copy(table_hbm.at[idx_v.at[j]], buf, sem)

        def gwait(idx_v, j, buf, sem):
            pltpu.make_async_copy(table_hbm.at[idx_v.at[j]], buf, sem).wait()

        def wb(j, buf, sem, out_hbm):
            pltpu.async_copy(buf, out_hbm.at[pl.ds(base + j * CH, CH)], sem)

        def wbwait(j, buf, sem, out_hbm):
            pltpu.make_async_copy(
                buf, out_hbm.at[pl.ds(base + j * CH, CH)], sem).wait()

        gth(sidx_v, 0, sbufa, ssema)
        gth(didx_v, 0, dbufa, dsema)
        gth(sidx_v, 1, sbufb, ssemb)
        gth(didx_v, 1, dbufb, dsemb)
        gwait(sidx_v, 0, sbufa, ssema)
        wb(0, sbufa, ssema, g1_hbm)
        gwait(didx_v, 0, dbufa, dsema)
        wb(0, dbufa, dsema, g2_hbm)
        gwait(sidx_v, 1, sbufb, ssemb)
        wb(1, sbufb, ssemb, g1_hbm)
        gwait(didx_v, 1, dbufb, dsemb)
        wb(1, dbufb, dsemb, g2_hbm)

        def body(i, carry):
            j0 = 2 * i
            j1 = 2 * i + 1
            wbwait(j0 - 2, sbufa, ssema, g1_hbm)
            gth(sidx_v, j0, sbufa, ssema)
            wbwait(j0 - 2, dbufa, dsema, g2_hbm)
            gth(didx_v, j0, dbufa, dsema)
            wbwait(j1 - 2, sbufb, ssemb, g1_hbm)
            gth(sidx_v, j1, sbufb, ssemb)
            wbwait(j1 - 2, dbufb, dsemb, g2_hbm)
            gth(didx_v, j1, dbufb, dsemb)
            gwait(sidx_v, j0, sbufa, ssema)
            wb(j0, sbufa, ssema, g1_hbm)
            gwait(didx_v, j0, dbufa, dsema)
            wb(j0, dbufa, dsema, g2_hbm)
            gwait(sidx_v, j1, sbufb, ssemb)
            wb(j1, sbufb, ssemb, g1_hbm)
            gwait(didx_v, j1, dbufb, dsemb)
            wb(j1, dbufb, dsemb, g2_hbm)
            return carry

        lax.fori_loop(1, NPAIR, body, 0)

        last = 2 * NPAIR - 2
        wbwait(last, sbufa, ssema, g1_hbm)
        wbwait(last, dbufa, dsema, g2_hbm)
        wbwait(last + 1, sbufb, ssemb, g1_hbm)
        wbwait(last + 1, dbufb, dsemb, g2_hbm)
        jlast = NCHUNK - 1
        gth(sidx_v, jlast, sbufa, ssema)
        gth(didx_v, jlast, dbufa, dsema)
        gwait(sidx_v, jlast, sbufa, ssema)
        pltpu.sync_copy(sbufa, g1_hbm.at[pl.ds(base + jlast * CH, CH)])
        gwait(didx_v, jlast, dbufa, dsema)
        pltpu.sync_copy(dbufa, g2_hbm.at[pl.ds(base + jlast * CH, CH)])

    return k(table, sidx3d, didx3d)


def _sc_scatter_add(new_edge, didx3d, zeros_stripe, phase):
    @functools.partial(
        pl.kernel,
        out_type=jax.ShapeDtypeStruct((NC, NPAD, LATENT), _f32),
        mesh=_sc_mesh(),
        name=f"sc_scatter_p{phase}",
        scratch_types=[
            pltpu.VMEM((NCHUNK, CH), jnp.int32),
            pltpu.VMEM((CH, LATENT), _f32),
            pltpu.VMEM((CH, LATENT), _f32),
            pltpu.VMEM_SHARED((NPAD, LATENT), _f32),
            pltpu.SemaphoreType.DMA,
            pltpu.SemaphoreType.DMA,
        ],
    )
    def k(ne_hbm, didx_hbm, zero_hbm, out_hbm, idx_v, rowa, rowb, acc,
          sema, semb):
        cid = lax.axis_index("c")
        sid = lax.axis_index("s")
        wid = sid * NC + cid
        base = phase * EPH + wid * EPW
        pltpu.sync_copy(didx_hbm.at[wid], idx_v)
        pltpu.sync_copy(zero_hbm, acc.at[pl.ds(sid * NPS, NPS)])
        plsc.subcore_barrier()

        def load(j, buf, sem):
            pltpu.async_copy(ne_hbm.at[pl.ds(base + j * CH, CH)], buf, sem)

        def lwait(j, buf, sem):
            pltpu.make_async_copy(
                ne_hbm.at[pl.ds(base + j * CH, CH)], buf, sem).wait()

        def sadd(j, buf, sem):
            pltpu.async_copy(buf, acc.at[idx_v.at[j]], sem, add=True)

        def swait(j, buf, sem):
            pltpu.make_async_copy(buf, acc.at[idx_v.at[j]], sem).wait()

        load(0, rowa, sema)
        load(1, rowb, semb)
        lwait(0, rowa, sema)
        sadd(0, rowa, sema)
        lwait(1, rowb, semb)
        sadd(1, rowb, semb)

        def body(i, carry):
            j0 = 2 * i
            j1 = 2 * i + 1
            swait(j0 - 2, rowa, sema)
            load(j0, rowa, sema)
            swait(j1 - 2, rowb, semb)
            load(j1, rowb, semb)
            lwait(j0, rowa, sema)
            sadd(j0, rowa, sema)
            lwait(j1, rowb, semb)
            sadd(j1, rowb, semb)
            return carry

        lax.fori_loop(1, NPAIR, body, 0)
        last = 2 * NPAIR - 2
        swait(last, rowa, sema)
        swait(last + 1, rowb, semb)
        jlast = NCHUNK - 1
        load(jlast, rowa, sema)
        lwait(jlast, rowa, sema)
        pltpu.sync_copy(rowa, acc.at[idx_v.at[jlast]], add=True)

        plsc.subcore_barrier()
        pltpu.sync_copy(acc.at[pl.ds(sid * NPS, NPS)],
                        out_hbm.at[cid, pl.ds(sid * NPS, NPS)])

    return k(new_edge, didx3d, zeros_stripe)


def _full(shape):
    nd = len(shape)
    return pl.BlockSpec(shape, lambda i: (0,) * nd)


def _rows(block, width):
    return pl.BlockSpec((block, width), lambda i: (i, 0))


def _rows_off(block, width, off):
    return pl.BlockSpec((block, width), lambda i, _o=off: (i + _o, 0))


def _call(body, grid, in_specs, out_specs, out_shape, args):
    return pl.pallas_call(
        body,
        grid=(grid,),
        in_specs=in_specs,
        out_specs=out_specs,
        out_shape=out_shape,
    )(*args)


def _w(x):
    return _full(x.shape)


def kernel(velocity, mesh_pos, params, node_type, srcs, dsts):
    p = params

    nm, ns = p["node_norm_mean"], p["node_norm_std"]
    ne_l = p["node_enc"]["layers"]
    w1n = ne_l[0]["W"] / ns[:, None]
    b1n = ne_l[0]["b"] - nm @ w1n
    wv = w1n[:2]
    woh = w1n[2:]

    em, es = p["edge_norm_mean"], p["edge_norm_std"]
    ee_l = p["edge_enc"]["layers"]
    w1e = ee_l[0]["W"] / es[:, None]
    b1e = ee_l[0]["b"] - em @ w1e

    dec_l = p["decoder"]["layers"]
    ostd, omean = p["out_norm_std"], p["out_norm_mean"]
    wd3 = dec_l[2]["W"] * ostd[None, :]
    bd3 = dec_l[2]["b"] * ostd + omean
    wd3p = jnp.zeros((LATENT, 128), _f32).at[:, :2].set(wd3)
    bd3p = jnp.zeros((128,), _f32).at[:2].set(bd3)

    def row(v):
        return v.reshape(1, -1)

    sidx4d = srcs.astype(jnp.int32).reshape(NPH, NW, NCHUNK, CH)
    didx4d = dsts.astype(jnp.int32).reshape(NPH, NW, NCHUNK, CH)
    sidx = [sidx4d[ph] for ph in range(NPH)]
    didx = [didx4d[ph] for ph in range(NPH)]
    zeros_stripe = jnp.zeros((NPS, LATENT), _f32)

    mp128 = jnp.zeros((N_NODES, LATENT), _f32).at[:, :2].set(mesh_pos)
    sd16 = [_sc_gather_pair(mp128, sidx[ph], didx[ph], ph)
            for ph in range(NPH)]

    ntf = node_type.astype(_f32).reshape(N_NODES, 1)
    node_lat = _call(
        _node_enc_kernel, N_NODES // NB,
        [_rows(NB, 2), _rows(NB, 1), _w(wv), _w(woh), _full((1, LATENT)),
         _w(ne_l[1]["W"]), _full((1, LATENT)), _w(ne_l[2]["W"]),
         _full((1, LATENT)), _full((1, LATENT)), _full((1, LATENT))],
        _rows(NB, LATENT),
        jax.ShapeDtypeStruct((N_NODES, LATENT), _f32),
        (velocity, ntf, wv, woh, row(b1n), ne_l[1]["W"], row(ne_l[1]["b"]),
         ne_l[2]["W"], row(ne_l[2]["b"]), row(p["node_enc"]["ln_scale"]),
         row(p["node_enc"]["ln_bias"])))

    edge_lat_ph0 = []
    for ph in range(NPH):
        s16, d16 = sd16[ph]
        off = ph * (EPH // EB)
        edge_lat_ph0.append(_call(
            _edge_enc_kernel, EPH // EB,
            [_rows_off(EB, LATENT, off), _rows_off(EB, LATENT, off), _w(w1e),
             _full((1, LATENT)), _w(ee_l[1]["W"]), _full((1, LATENT)),
             _w(ee_l[2]["W"]), _full((1, LATENT)), _full((1, LATENT)),
             _full((1, LATENT))],
            _rows_off(EB, LATENT, off),
            jax.ShapeDtypeStruct((N_EDGES, LATENT), _f32),
            (s16, d16, w1e, row(b1e), ee_l[1]["W"], row(ee_l[1]["b"]),
             ee_l[2]["W"], row(ee_l[2]["b"]), row(p["edge_enc"]["ln_scale"]),
             row(p["edge_enc"]["ln_bias"]))))

    edge_lat_ph = edge_lat_ph0
    for step in p["mp"]:
        el_ = step["edge"]["layers"]
        nl_ = step["node"]["layers"]
        w1 = el_[0]["W"]

        gs = [_sc_gather_pair(node_lat, sidx[ph], didx[ph], ph)
              for ph in range(NPH)]

        aggs = []
        new_el_ph = []
        for ph in range(NPH):
            g1, g2 = gs[ph]
            off = ph * (EPH // EB)
            ne_el = _call(
                _edge_step_kernel, EPH // EB,
                [_rows_off(EB, LATENT, off), _rows_off(EB, LATENT, off),
                 _rows_off(EB, LATENT, off),
                 _w(w1), _full((1, LATENT)), _w(el_[1]["W"]),
                 _full((1, LATENT)), _w(el_[2]["W"]), _full((1, LATENT)),
                 _full((1, LATENT)), _full((1, LATENT))],
                [_rows_off(EB, LATENT, off), _rows_off(EB, LATENT, off)],
                [jax.ShapeDtypeStruct((N_EDGES, LATENT), _f32),
                 jax.ShapeDtypeStruct((N_EDGES, LATENT), _f32)],
                (edge_lat_ph[ph], g1, g2, w1, row(el_[0]["b"]), el_[1]["W"],
                 row(el_[1]["b"]), el_[2]["W"], row(el_[2]["b"]),
                 row(step["edge"]["ln_scale"]), row(step["edge"]["ln_bias"])))
            new_edge, elo = ne_el
            new_el_ph.append(elo)
            aggs.append(_sc_scatter_add(new_edge, didx[ph], zeros_stripe, ph))
        edge_lat_ph = new_el_ph

        wn1 = nl_[0]["W"]
        node_lat = _call(
            _node_step_kernel, N_NODES // NB,
            [_rows(NB, LATENT),
             pl.BlockSpec((NC, NB, LATENT), lambda i: (0, i, 0)),
             pl.BlockSpec((NC, NB, LATENT), lambda i: (0, i, 0)), _w(wn1),
             _full((1, LATENT)), _w(nl_[1]["W"]), _full((1, LATENT)),
             _w(nl_[2]["W"]), _full((1, LATENT)), _full((1, LATENT)),
             _full((1, LATENT))],
            _rows(NB, LATENT),
            jax.ShapeDtypeStruct((N_NODES, LATENT), _f32),
            (node_lat, aggs[0], aggs[1], wn1, row(nl_[0]["b"]), nl_[1]["W"],
             row(nl_[1]["b"]), nl_[2]["W"], row(nl_[2]["b"]),
             row(step["node"]["ln_scale"]), row(step["node"]["ln_bias"])))

    outp = _call(
        _decoder_kernel, N_NODES // NB,
        [_rows(NB, LATENT), _w(dec_l[0]["W"]), _full((1, LATENT)),
         _w(dec_l[1]["W"]), _full((1, LATENT)), _w(wd3p), _full((1, 128))],
        _rows(NB, 128),
        jax.ShapeDtypeStruct((N_NODES, 128), _f32),
        (node_lat, dec_l[0]["W"], row(dec_l[0]["b"]), dec_l[1]["W"],
         row(dec_l[1]["b"]), wd3p, row(bd3p)))
    return outp[:, :2]
